# Initial kernel scaffold; baseline (speedup 1.0000x reference)
import jax
import jax.numpy as jnp
from jax.experimental import pallas as pl


def kernel(indices, weight):
    raise NotImplementedError("write your pallas kernel here")


# speedup vs baseline: 1.3784x; 1.3784x over previous
import functools

import jax
import jax.numpy as jnp
from jax import lax
from jax.experimental import pallas as pl
from jax.experimental.pallas import tpu as pltpu
from jax.experimental.pallas import tpu_sc as plsc

NUM_ROWS = 1000001
DIM = 32
B_TOTAL = 4096 * 200
NC, NS = 2, 16
NW = NC * NS
CHUNK = 128
N_CHUNKS = B_TOTAL // (NW * CHUNK)
B_PER_W = N_CHUNKS * CHUNK


def _body(idx_hbm, table_hbm, out_hbm, idx_v, rows_v, sem):
    wid = lax.axis_index("s") * NC + lax.axis_index("c")
    base = wid * B_PER_W
    pltpu.sync_copy(idx_hbm.at[wid], idx_v)

    def step(j):
        pltpu.async_copy(table_hbm.at[idx_v.at[j]], rows_v, sem).wait()
        pltpu.sync_copy(rows_v, out_hbm.at[pl.ds(base + j * CHUNK, CHUNK)])

    pl.loop(0, N_CHUNKS)(step)


@jax.jit
def _gather(indices_blocked, weight):
    mesh = plsc.VectorSubcoreMesh(core_axis_name="c", subcore_axis_name="s")
    flat = pl.kernel(
        _body,
        out_type=jax.ShapeDtypeStruct((B_TOTAL, DIM), jnp.float32),
        mesh=mesh,
        scratch_types=[
            pltpu.VMEM((N_CHUNKS, CHUNK), jnp.int32),
            pltpu.VMEM((CHUNK, DIM), jnp.float32),
            pltpu.SemaphoreType.DMA,
        ],
        compiler_params=pltpu.CompilerParams(use_tc_tiling_on_sc=False),
    )(indices_blocked, weight)
    return flat


def kernel(indices, weight):
    idx_blocked = indices.reshape(NW, N_CHUNKS, CHUNK)
    flat = _gather(idx_blocked, weight)
    return flat.reshape(indices.shape + (DIM,))


# speedup vs baseline: 1.5859x; 1.1506x over previous
import jax
import jax.numpy as jnp
from jax import lax
from jax.experimental import pallas as pl
from jax.experimental.pallas import tpu as pltpu
from jax.experimental.pallas import tpu_sc as plsc

NUM_ROWS = 1000001
DIM = 32
B_TOTAL = 4096 * 200
NC, NS = 2, 16
NW = NC * NS
CHUNK = 128
N_CHUNKS = B_TOTAL // (NW * CHUNK)
B_PER_W = N_CHUNKS * CHUNK
NBUF = 8
N_GROUPS = N_CHUNKS // NBUF


def _body(idx_hbm, table_hbm, out_hbm, idx_v, *scratch):
    bufs = scratch[:NBUF]
    sems = scratch[NBUF:]
    wid = lax.axis_index("s") * NC + lax.axis_index("c")
    base = wid * B_PER_W
    pltpu.sync_copy(idx_hbm.at[wid], idx_v)

    for b in range(NBUF):
        pltpu.make_async_copy(table_hbm.at[idx_v.at[b]], bufs[b], sems[b]).start()

    def group(g):
        j0 = g * NBUF
        for b in range(NBUF):
            j = j0 + b
            pltpu.make_async_copy(
                table_hbm.at[pl.ds(0, CHUNK)], bufs[b], sems[b]
            ).wait()
            pltpu.sync_copy(bufs[b], out_hbm.at[pl.ds(base + j * CHUNK, CHUNK)])
            nxt = j + NBUF

            @pl.when(nxt < N_CHUNKS)
            def _():
                pltpu.make_async_copy(
                    table_hbm.at[idx_v.at[nxt]], bufs[b], sems[b]
                ).start()

    pl.loop(0, N_GROUPS)(group)


@jax.jit
def _gather(indices_blocked, weight):
    mesh = plsc.VectorSubcoreMesh(core_axis_name="c", subcore_axis_name="s")
    flat = pl.kernel(
        _body,
        out_type=jax.ShapeDtypeStruct((B_TOTAL, DIM), jnp.float32),
        mesh=mesh,
        scratch_types=[pltpu.VMEM((N_CHUNKS, CHUNK), jnp.int32)]
        + [pltpu.VMEM((CHUNK, DIM), jnp.float32) for _ in range(NBUF)]
        + [pltpu.SemaphoreType.DMA for _ in range(NBUF)],
        compiler_params=pltpu.CompilerParams(use_tc_tiling_on_sc=False),
    )(indices_blocked, weight)
    return flat


def kernel(indices, weight):
    idx_blocked = indices.reshape(NW, N_CHUNKS, CHUNK)
    flat = _gather(idx_blocked, weight)
    return flat.reshape(indices.shape + (DIM,))
